# baseline (speedup 1.0000x reference)
import jax
import jax.numpy as jnp
from jax import lax
from jax.experimental import pallas as pl

B, N, D, M, K_TXT = 8, 196, 512, 16, 8
P = B * N
KEEP = max(int(D * 0.1), 1)
PB = 8
NG = 4
PBS = PB * NG
GB = P // PBS
GROUPS = tuple((g * PB, PB) for g in range(NG))
PA = 392
GA = P // PA

_INV_SQRT2 = 0.7071067811865476


def _mlp_kernel(f_ref, tf_ref, w1_ref, b1_ref, g_ref, bt_ref, w2_ref,
                b2_ref, qp_ref):
    tf = tf_ref[...]
    tbar = jnp.mean(tf, axis=0, keepdims=True)
    f = f_ref[...]
    acc = jnp.zeros_like(f)
    for k in range(K_TXT):
        tk = tf[k:k + 1, :] - tbar
        x = f * tk
        h = lax.dot_general(x, w1_ref[...], (((1,), (1,)), ((), ())),
                            preferred_element_type=jnp.float32,
                        precision=lax.Precision.HIGHEST)
        h = h + b1_ref[...]
        mu = jnp.mean(h, axis=-1, keepdims=True)
        hc = h - mu
        var = jnp.mean(hc * hc, axis=-1, keepdims=True)
        hn = hc * lax.rsqrt(var + 1e-5) * g_ref[...] + bt_ref[...]
        acc = acc + 0.5 * hn * (1.0 + lax.erf(hn * _INV_SQRT2))
    qp = lax.dot_general(acc * (1.0 / K_TXT), w2_ref[...],
                         (((1,), (1,)), ((), ())),
                         preferred_element_type=jnp.float32,
                        precision=lax.Precision.HIGHEST)
    qp_ref[...] = qp + b2_ref[...]


def _topk_kernel(f_ref, tmpl_ref, qp_ref, e_ref, out_ref):
    ng = len(GROUPS)
    xs_g = []
    gb_g = []
    for start, npat in GROUPS:
        fg = f_ref[0, start:start + npat, :]
        fr = lax.dot_general(fg, e_ref[:npat, :], (((0,), (0,)), ((), ())),
                             preferred_element_type=jnp.float32,
                             precision=lax.Precision.HIGHEST)
        x = fr * tmpl_ref[...]
        xs_g.append(x)
        gb_g.append(lax.bitcast_convert_type(jnp.abs(x), jnp.int32))
    gbits = jnp.stack(gb_g, axis=0)
    ones_b = jnp.ones((ng, 1, D), jnp.bfloat16)
    lo = jnp.zeros((ng, 1, PB * M), jnp.int32)
    hi = jnp.full((ng, 1, PB * M), 0x7F800000, jnp.int32)

    def body(_, lh):
        lo_, hi_ = lh
        mid = lo_ + lax.shift_right_arithmetic(hi_ - lo_, 1)
        sel = jnp.where(gbits >= mid, 1.0, 0.0).astype(jnp.bfloat16)
        cnt = lax.dot_general(ones_b, sel, (((2,), (1,)), ((0,), (0,))),
                              preferred_element_type=jnp.float32)
        pred = cnt >= float(KEEP)
        return (jnp.where(pred, mid, lo_), jnp.where(pred, hi_, mid))

    lo, hi = lax.fori_loop(0, 31, body, (lo, hi))
    for g, (start, npat) in enumerate(GROUPS):
        masked = jnp.where(gb_g[g] >= lo[g], xs_g[g], 0.0)
        for p in range(npat):
            xs = lax.slice(masked, (0, M * p), (D, M * p + M))
            q = qp_ref[0, start + p:start + p + 1, :]
            out_ref[0, start + p] = xs * q


def kernel(F_clean, text_features, W1, b1, ln_g, ln_b, W2, b2, templates):
    f2 = F_clean.reshape(P, D)
    b1r = b1.reshape(1, D)
    gr = ln_g.reshape(1, D)
    btr = ln_b.reshape(1, D)
    b2r = b2.reshape(1, M)
    qp = pl.pallas_call(
        _mlp_kernel,
        grid=(GA,),
        in_specs=[
            pl.BlockSpec((PA, D), lambda i: (i, 0)),
            pl.BlockSpec((K_TXT, D), lambda i: (0, 0)),
            pl.BlockSpec((D, D), lambda i: (0, 0)),
            pl.BlockSpec((1, D), lambda i: (0, 0)),
            pl.BlockSpec((1, D), lambda i: (0, 0)),
            pl.BlockSpec((1, D), lambda i: (0, 0)),
            pl.BlockSpec((M, D), lambda i: (0, 0)),
            pl.BlockSpec((1, M), lambda i: (0, 0)),
        ],
        out_specs=pl.BlockSpec((PA, M), lambda i: (i, 0)),
        out_shape=jax.ShapeDtypeStruct((P, M), jnp.float32),
    )(f2, text_features, W1, b1r, gr, btr, W2, b2r)

    qp3 = qp.reshape(GB, PBS, M)
    f3 = f2.reshape(GB, PBS, D)
    tmpl_tiled = jnp.tile(templates.T, (1, PB))
    e_rep = jnp.repeat(jnp.eye(PB, dtype=jnp.float32), M, axis=1)

    out4 = pl.pallas_call(
        _topk_kernel,
        grid=(GB,),
        in_specs=[
            pl.BlockSpec((1, PBS, D), lambda i: (i, 0, 0)),
            pl.BlockSpec((D, PB * M), lambda i: (0, 0)),
            pl.BlockSpec((1, PBS, M), lambda i: (i, 0, 0)),
            pl.BlockSpec((PB, PB * M), lambda i: (0, 0)),
        ],
        out_specs=pl.BlockSpec((1, PBS, D, M), lambda i: (i, 0, 0, 0)),
        out_shape=jax.ShapeDtypeStruct((GB, PBS, D, M), jnp.float32),
    )(f3, tmpl_tiled, qp3, e_rep)

    return out4.reshape(B, N, D, M)


# speedup vs baseline: 1.2762x; 1.2762x over previous
import jax
import jax.numpy as jnp
from jax import lax
from jax.experimental import pallas as pl

B, N, D, M, K_TXT = 8, 196, 512, 16, 8
P = B * N
KEEP = max(int(D * 0.1), 1)
PB = 8
NG = 4
PBS = PB * NG
GB = P // PBS
GROUPS = tuple((g * PB, PB) for g in range(NG))
PA = 392
GA = P // PA

_INV_SQRT2 = 0.7071067811865476


def _mlp_kernel(f_ref, tf_ref, w1_ref, b1_ref, g_ref, bt_ref, w2_ref,
                b2_ref, qp_ref):
    tf = tf_ref[...]
    tbar = jnp.mean(tf, axis=0, keepdims=True)
    f = f_ref[...]
    acc = jnp.zeros_like(f)
    for k in range(K_TXT):
        tk = tf[k:k + 1, :] - tbar
        x = f * tk
        h = lax.dot_general(x, w1_ref[...], (((1,), (1,)), ((), ())),
                            preferred_element_type=jnp.float32,
                        precision=lax.Precision.HIGHEST)
        h = h + b1_ref[...]
        mu = jnp.mean(h, axis=-1, keepdims=True)
        hc = h - mu
        var = jnp.mean(hc * hc, axis=-1, keepdims=True)
        hn = hc * lax.rsqrt(var + 1e-5) * g_ref[...] + bt_ref[...]
        acc = acc + 0.5 * hn * (1.0 + lax.erf(hn * _INV_SQRT2))
    qp = lax.dot_general(acc * (1.0 / K_TXT), w2_ref[...],
                         (((1,), (1,)), ((), ())),
                         preferred_element_type=jnp.float32,
                        precision=lax.Precision.HIGHEST)
    qp_ref[...] = qp + b2_ref[...]


def _topk_kernel(f_ref, tmpl_ref, qp_ref, e_ref, out_ref):
    ng = len(GROUPS)
    xs_g = []
    gb_g = []
    for start, npat in GROUPS:
        fg = f_ref[0, start:start + npat, :]
        fr = lax.dot_general(fg, e_ref[:npat, :], (((0,), (0,)), ((), ())),
                             preferred_element_type=jnp.float32,
                             precision=lax.Precision.HIGHEST)
        x = fr * tmpl_ref[...]
        xs_g.append(x)
        gb_g.append(lax.bitcast_convert_type(jnp.abs(x), jnp.int32))
    gbits = jnp.stack(gb_g, axis=0)
    lo = jnp.zeros((ng, 1, PB * M), jnp.int32)
    hi = jnp.full((ng, 1, PB * M), 0x7F800000, jnp.int32)

    def body(_, lh):
        lo_, hi_ = lh
        mid = lo_ + lax.shift_right_arithmetic(hi_ - lo_, 1)
        cnt = jnp.sum((gbits >= mid).astype(jnp.int32), axis=1,
                      keepdims=True)
        pred = cnt >= KEEP
        return (jnp.where(pred, mid, lo_), jnp.where(pred, hi_, mid))

    lo, hi = lax.fori_loop(0, 31, body, (lo, hi), unroll=4)
    for g, (start, npat) in enumerate(GROUPS):
        masked = jnp.where(gb_g[g] >= lo[g], xs_g[g], 0.0)
        for p in range(npat):
            xs = lax.slice(masked, (0, M * p), (D, M * p + M))
            q = qp_ref[0, start + p:start + p + 1, :]
            out_ref[0, start + p] = xs * q


def kernel(F_clean, text_features, W1, b1, ln_g, ln_b, W2, b2, templates):
    f2 = F_clean.reshape(P, D)
    b1r = b1.reshape(1, D)
    gr = ln_g.reshape(1, D)
    btr = ln_b.reshape(1, D)
    b2r = b2.reshape(1, M)
    qp = pl.pallas_call(
        _mlp_kernel,
        grid=(GA,),
        in_specs=[
            pl.BlockSpec((PA, D), lambda i: (i, 0)),
            pl.BlockSpec((K_TXT, D), lambda i: (0, 0)),
            pl.BlockSpec((D, D), lambda i: (0, 0)),
            pl.BlockSpec((1, D), lambda i: (0, 0)),
            pl.BlockSpec((1, D), lambda i: (0, 0)),
            pl.BlockSpec((1, D), lambda i: (0, 0)),
            pl.BlockSpec((M, D), lambda i: (0, 0)),
            pl.BlockSpec((1, M), lambda i: (0, 0)),
        ],
        out_specs=pl.BlockSpec((PA, M), lambda i: (i, 0)),
        out_shape=jax.ShapeDtypeStruct((P, M), jnp.float32),
    )(f2, text_features, W1, b1r, gr, btr, W2, b2r)

    qp3 = qp.reshape(GB, PBS, M)
    f3 = f2.reshape(GB, PBS, D)
    tmpl_tiled = jnp.tile(templates.T, (1, PB))
    e_rep = jnp.repeat(jnp.eye(PB, dtype=jnp.float32), M, axis=1)

    out4 = pl.pallas_call(
        _topk_kernel,
        grid=(GB,),
        in_specs=[
            pl.BlockSpec((1, PBS, D), lambda i: (i, 0, 0)),
            pl.BlockSpec((D, PB * M), lambda i: (0, 0)),
            pl.BlockSpec((1, PBS, M), lambda i: (i, 0, 0)),
            pl.BlockSpec((PB, PB * M), lambda i: (0, 0)),
        ],
        out_specs=pl.BlockSpec((1, PBS, D, M), lambda i: (i, 0, 0, 0)),
        out_shape=jax.ShapeDtypeStruct((GB, PBS, D, M), jnp.float32),
    )(f3, tmpl_tiled, qp3, e_rep)

    return out4.reshape(B, N, D, M)


# speedup vs baseline: 1.3159x; 1.0311x over previous
import jax
import jax.numpy as jnp
from jax import lax
from jax.experimental import pallas as pl

B, N, D, M, K_TXT = 8, 196, 512, 16, 8
P = B * N
KEEP = max(int(D * 0.1), 1)
PB = 8
NG = 4
PBS = PB * NG
GB = P // PBS
GROUPS = tuple((g * PB, PB) for g in range(NG))
PA = 392
GA = P // PA

_INV_SQRT2 = 0.7071067811865476


def _mlp_kernel(f_ref, tf_ref, w1_ref, b1_ref, g_ref, bt_ref, w2_ref,
                b2_ref, qp_ref):
    tf = tf_ref[...]
    tbar = jnp.mean(tf, axis=0, keepdims=True)
    f = f_ref[...]
    acc = jnp.zeros_like(f)
    for k in range(K_TXT):
        tk = tf[k:k + 1, :] - tbar
        x = f * tk
        h = lax.dot_general(x, w1_ref[...], (((1,), (1,)), ((), ())),
                            preferred_element_type=jnp.float32,
                        precision=lax.Precision.HIGHEST)
        h = h + b1_ref[...]
        mu = jnp.mean(h, axis=-1, keepdims=True)
        hc = h - mu
        var = jnp.mean(hc * hc, axis=-1, keepdims=True)
        hn = hc * lax.rsqrt(var + 1e-5) * g_ref[...] + bt_ref[...]
        acc = acc + 0.5 * hn * (1.0 + lax.erf(hn * _INV_SQRT2))
    qp = lax.dot_general(acc * (1.0 / K_TXT), w2_ref[...],
                         (((1,), (1,)), ((), ())),
                         preferred_element_type=jnp.float32,
                        precision=lax.Precision.HIGHEST)
    qp_ref[...] = qp + b2_ref[...]


def _topk_kernel(f_ref, tmpl_ref, qp_ref, e_ref, out_ref):
    ng = len(GROUPS)
    xs_g = []
    gb_g = []
    for start, npat in GROUPS:
        fg = f_ref[0, start:start + npat, :]
        fr = lax.dot_general(fg, e_ref[:npat, :], (((0,), (0,)), ((), ())),
                             preferred_element_type=jnp.float32,
                             precision=lax.Precision.HIGHEST)
        x = fr * tmpl_ref[...]
        xs_g.append(x)
        gb_g.append(lax.bitcast_convert_type(jnp.abs(x), jnp.int32))
    gbits = jnp.stack(gb_g, axis=0)
    lo = jnp.zeros((ng, 1, PB * M), jnp.int32)
    hi = jnp.full((ng, 1, PB * M), 0x7F800000, jnp.int32)

    def body(_, lh):
        lo_, hi_ = lh
        mid = lo_ + lax.shift_right_arithmetic(hi_ - lo_, 1)
        cnt = jnp.sum((gbits >= mid).astype(jnp.int32), axis=1,
                      keepdims=True)
        pred = cnt >= KEEP
        return (jnp.where(pred, mid, lo_), jnp.where(pred, hi_, mid))

    lo, hi = lax.fori_loop(0, 31, body, (lo, hi), unroll=31)
    for g, (start, npat) in enumerate(GROUPS):
        masked = jnp.where(gb_g[g] >= lo[g], xs_g[g], 0.0)
        for p in range(npat):
            xs = lax.slice(masked, (0, M * p), (D, M * p + M))
            q = qp_ref[0, start + p:start + p + 1, :]
            out_ref[0, start + p] = xs * q


def kernel(F_clean, text_features, W1, b1, ln_g, ln_b, W2, b2, templates):
    f2 = F_clean.reshape(P, D)
    b1r = b1.reshape(1, D)
    gr = ln_g.reshape(1, D)
    btr = ln_b.reshape(1, D)
    b2r = b2.reshape(1, M)
    qp = pl.pallas_call(
        _mlp_kernel,
        grid=(GA,),
        in_specs=[
            pl.BlockSpec((PA, D), lambda i: (i, 0)),
            pl.BlockSpec((K_TXT, D), lambda i: (0, 0)),
            pl.BlockSpec((D, D), lambda i: (0, 0)),
            pl.BlockSpec((1, D), lambda i: (0, 0)),
            pl.BlockSpec((1, D), lambda i: (0, 0)),
            pl.BlockSpec((1, D), lambda i: (0, 0)),
            pl.BlockSpec((M, D), lambda i: (0, 0)),
            pl.BlockSpec((1, M), lambda i: (0, 0)),
        ],
        out_specs=pl.BlockSpec((PA, M), lambda i: (i, 0)),
        out_shape=jax.ShapeDtypeStruct((P, M), jnp.float32),
    )(f2, text_features, W1, b1r, gr, btr, W2, b2r)

    qp3 = qp.reshape(GB, PBS, M)
    f3 = f2.reshape(GB, PBS, D)
    tmpl_tiled = jnp.tile(templates.T, (1, PB))
    e_rep = jnp.repeat(jnp.eye(PB, dtype=jnp.float32), M, axis=1)

    out4 = pl.pallas_call(
        _topk_kernel,
        grid=(GB,),
        in_specs=[
            pl.BlockSpec((1, PBS, D), lambda i: (i, 0, 0)),
            pl.BlockSpec((D, PB * M), lambda i: (0, 0)),
            pl.BlockSpec((1, PBS, M), lambda i: (i, 0, 0)),
            pl.BlockSpec((PB, PB * M), lambda i: (0, 0)),
        ],
        out_specs=pl.BlockSpec((1, PBS, D, M), lambda i: (i, 0, 0, 0)),
        out_shape=jax.ShapeDtypeStruct((GB, PBS, D, M), jnp.float32),
    )(f3, tmpl_tiled, qp3, e_rep)

    return out4.reshape(B, N, D, M)


# speedup vs baseline: 1.4054x; 1.0680x over previous
import jax
import jax.numpy as jnp
from jax import lax
from jax.experimental import pallas as pl

B, N, D, M, K_TXT = 8, 196, 512, 16, 8
P = B * N
KEEP = max(int(D * 0.1), 1)
PB = 8
NG = 7
PBS = PB * NG
GB = P // PBS
GROUPS = tuple((g * PB, PB) for g in range(NG))
PA = 392
GA = P // PA

_INV_SQRT2 = 0.7071067811865476


def _mlp_kernel(f_ref, tf_ref, w1_ref, b1_ref, g_ref, bt_ref, w2_ref,
                b2_ref, qp_ref):
    tf = tf_ref[...]
    tbar = jnp.mean(tf, axis=0, keepdims=True)
    f = f_ref[...]
    acc = jnp.zeros_like(f)
    for k in range(K_TXT):
        tk = tf[k:k + 1, :] - tbar
        x = f * tk
        h = lax.dot_general(x, w1_ref[...], (((1,), (1,)), ((), ())),
                            preferred_element_type=jnp.float32,
                        precision=lax.Precision.HIGHEST)
        h = h + b1_ref[...]
        mu = jnp.mean(h, axis=-1, keepdims=True)
        hc = h - mu
        var = jnp.mean(hc * hc, axis=-1, keepdims=True)
        hn = hc * lax.rsqrt(var + 1e-5) * g_ref[...] + bt_ref[...]
        acc = acc + 0.5 * hn * (1.0 + lax.erf(hn * _INV_SQRT2))
    qp = lax.dot_general(acc * (1.0 / K_TXT), w2_ref[...],
                         (((1,), (1,)), ((), ())),
                         preferred_element_type=jnp.float32,
                        precision=lax.Precision.HIGHEST)
    qp_ref[...] = qp + b2_ref[...]


def _topk_kernel(f_ref, tmpl_ref, qp_ref, e_ref, out_ref):
    ng = len(GROUPS)
    xs_g = []
    gb_g = []
    for start, npat in GROUPS:
        fg = f_ref[0, start:start + npat, :]
        fr = lax.dot_general(fg, e_ref[:npat, :], (((0,), (0,)), ((), ())),
                             preferred_element_type=jnp.float32,
                             precision=lax.Precision.HIGHEST)
        x = fr * tmpl_ref[...]
        xs_g.append(x)
        gb_g.append(lax.bitcast_convert_type(jnp.abs(x), jnp.int32))
    gbits = jnp.stack(gb_g, axis=0)
    lo = jnp.zeros((ng, 1, PB * M), jnp.int32)
    hi = jnp.full((ng, 1, PB * M), 0x7F800000, jnp.int32)

    def body(_, lh):
        lo_, hi_ = lh
        mid = lo_ + lax.shift_right_arithmetic(hi_ - lo_, 1)
        cnt = jnp.sum((gbits >= mid).astype(jnp.int32), axis=1,
                      keepdims=True)
        pred = cnt >= KEEP
        return (jnp.where(pred, mid, lo_), jnp.where(pred, hi_, mid))

    lo, hi = lax.fori_loop(0, 31, body, (lo, hi), unroll=31)
    for g, (start, npat) in enumerate(GROUPS):
        masked = jnp.where(gb_g[g] >= lo[g], xs_g[g], 0.0)
        for p in range(npat):
            xs = lax.slice(masked, (0, M * p), (D, M * p + M))
            q = qp_ref[0, start + p:start + p + 1, :]
            out_ref[0, start + p] = xs * q


def kernel(F_clean, text_features, W1, b1, ln_g, ln_b, W2, b2, templates):
    f2 = F_clean.reshape(P, D)
    b1r = b1.reshape(1, D)
    gr = ln_g.reshape(1, D)
    btr = ln_b.reshape(1, D)
    b2r = b2.reshape(1, M)
    qp = pl.pallas_call(
        _mlp_kernel,
        grid=(GA,),
        in_specs=[
            pl.BlockSpec((PA, D), lambda i: (i, 0)),
            pl.BlockSpec((K_TXT, D), lambda i: (0, 0)),
            pl.BlockSpec((D, D), lambda i: (0, 0)),
            pl.BlockSpec((1, D), lambda i: (0, 0)),
            pl.BlockSpec((1, D), lambda i: (0, 0)),
            pl.BlockSpec((1, D), lambda i: (0, 0)),
            pl.BlockSpec((M, D), lambda i: (0, 0)),
            pl.BlockSpec((1, M), lambda i: (0, 0)),
        ],
        out_specs=pl.BlockSpec((PA, M), lambda i: (i, 0)),
        out_shape=jax.ShapeDtypeStruct((P, M), jnp.float32),
    )(f2, text_features, W1, b1r, gr, btr, W2, b2r)

    qp3 = qp.reshape(GB, PBS, M)
    f3 = f2.reshape(GB, PBS, D)
    tmpl_tiled = jnp.tile(templates.T, (1, PB))
    e_rep = jnp.repeat(jnp.eye(PB, dtype=jnp.float32), M, axis=1)

    out4 = pl.pallas_call(
        _topk_kernel,
        grid=(GB,),
        in_specs=[
            pl.BlockSpec((1, PBS, D), lambda i: (i, 0, 0)),
            pl.BlockSpec((D, PB * M), lambda i: (0, 0)),
            pl.BlockSpec((1, PBS, M), lambda i: (i, 0, 0)),
            pl.BlockSpec((PB, PB * M), lambda i: (0, 0)),
        ],
        out_specs=pl.BlockSpec((1, PBS, D, M), lambda i: (i, 0, 0, 0)),
        out_shape=jax.ShapeDtypeStruct((GB, PBS, D, M), jnp.float32),
    )(f3, tmpl_tiled, qp3, e_rep)

    return out4.reshape(B, N, D, M)
